# baseline (speedup 1.0000x reference)
import jax
import jax.numpy as jnp
import numpy as np
from jax import lax
from jax.experimental import pallas as pl
from jax.experimental.pallas import tpu as pltpu

N_MOSSY = 1024
N_GRANULE = 65536
K = int(N_GRANULE * 0.02)

RB = 512
GRID = N_GRANULE // RB
INT_MIN = np.int32(-(2 ** 31))


def _sortable_key(y):
    s = lax.bitcast_convert_type(y, jnp.int32)
    return jnp.where(s < 0, ~(s ^ INT_MIN), s)


def _body(u_ref, w_ref, m_ref, thr_ref, out_ref):
    b = pl.program_id(0)
    masked = w_ref[...] * m_ref[...]
    y_blk = lax.dot_general(u_ref[...], masked, (((1,), (1,)), ((), ())),
                            preferred_element_type=jnp.float32)
    out_ref[pl.ds(b, 1), :] = y_blk

    @pl.when(b == GRID - 1)
    def _select():
        y = out_ref[...]
        key = _sortable_key(y)

        def bit_step(i, t_u):
            cand = t_u | (jnp.int32(1) << (31 - i))
            d = cand ^ INT_MIN
            c = jnp.sum((key >= d).astype(jnp.int32))
            return jnp.where(c >= K, cand, t_u)

        t_u = lax.fori_loop(0, 32, bit_step, jnp.int32(0))
        t = t_u ^ INT_MIN

        gt = key > t
        c_gt = jnp.sum(gt.astype(jnp.int32))
        m = K - c_gt
        eq = key == t

        ri = lax.broadcasted_iota(jnp.int32, (GRID, RB), 0)
        ci = lax.broadcasted_iota(jnp.int32, (GRID, RB), 1)
        idx = ri * RB + ci

        def idx_step(i, x):
            cand = x | (jnp.int32(1) << (16 - i))
            c = jnp.sum((eq & (idx < cand)).astype(jnp.int32))
            return jnp.where(c < m, cand, x)

        x = lax.fori_loop(0, 17, idx_step, jnp.int32(0))
        sel = gt | (eq & (idx < x + 1))

        thr = thr_ref[0]
        out_ref[...] = jnp.where(sel, jnp.maximum(y - thr, 0.0), 0.0)


@jax.jit
def kernel(mossy_input, connectivity_mask, weights, threshold):
    u2 = mossy_input.reshape(1, N_MOSSY)
    thr = threshold.reshape(1)

    out = pl.pallas_call(
        _body,
        grid=(GRID,),
        in_specs=[
            pl.BlockSpec((1, N_MOSSY), lambda b: (0, 0)),
            pl.BlockSpec((RB, N_MOSSY), lambda b: (b, 0)),
            pl.BlockSpec((RB, N_MOSSY), lambda b: (b, 0)),
            pl.BlockSpec(memory_space=pltpu.SMEM),
        ],
        out_specs=pl.BlockSpec((GRID, RB), lambda b: (0, 0)),
        out_shape=jax.ShapeDtypeStruct((GRID, RB), jnp.float32),
        compiler_params=pltpu.CompilerParams(
            dimension_semantics=("arbitrary",),
        ),
    )(u2, weights, connectivity_mask, thr)
    return out.reshape(N_GRANULE)


# speedup vs baseline: 1.0996x; 1.0996x over previous
import jax
import jax.numpy as jnp
import numpy as np
from jax import lax
from jax.experimental import pallas as pl
from jax.experimental.pallas import tpu as pltpu

N_MOSSY = 1024
N_GRANULE = 65536
K = int(N_GRANULE * 0.02)

RB = 2048
GRID = N_GRANULE // RB
INT_MIN = np.int32(-(2 ** 31))


def _sortable_key(y):
    s = lax.bitcast_convert_type(y, jnp.int32)
    return jnp.where(s < 0, ~(s ^ INT_MIN), s)


def _body(u_ref, w_ref, m_ref, thr_ref, out_ref):
    b = pl.program_id(0)
    masked = w_ref[...] * m_ref[...]
    y_blk = lax.dot_general(u_ref[...], masked, (((1,), (1,)), ((), ())),
                            preferred_element_type=jnp.float32)
    out_ref[pl.ds(b, 1), :] = y_blk

    @pl.when(b == GRID - 1)
    def _select():
        y = out_ref[...]
        key = _sortable_key(y)

        def bit_step(i, t_u):
            cand = t_u | (jnp.int32(1) << (31 - i))
            d = cand ^ INT_MIN
            c = jnp.sum((key >= d).astype(jnp.int32))
            return jnp.where(c >= K, cand, t_u)

        t_u = lax.fori_loop(0, 32, bit_step, jnp.int32(0))
        t = t_u ^ INT_MIN

        gt = key > t
        eq = key == t
        c_gt = jnp.sum(gt.astype(jnp.int32))
        c_eq = jnp.sum(eq.astype(jnp.int32))
        m = K - c_gt

        ri = lax.broadcasted_iota(jnp.int32, (GRID, RB), 0)
        ci = lax.broadcasted_iota(jnp.int32, (GRID, RB), 1)
        idx = ri * RB + ci

        def _all_ties(_):
            return jnp.int32(N_GRANULE)

        def _search_ties(_):
            def idx_step(i, x):
                cand = x | (jnp.int32(1) << (16 - i))
                c = jnp.sum((eq & (idx < cand)).astype(jnp.int32))
                return jnp.where(c < m, cand, x)

            return lax.fori_loop(0, 17, idx_step, jnp.int32(0))

        x = lax.cond(c_eq == m, _all_ties, _search_ties, jnp.int32(0))
        sel = gt | (eq & (idx < x + 1))

        thr = thr_ref[0]
        out_ref[...] = jnp.where(sel, jnp.maximum(y - thr, 0.0), 0.0)


@jax.jit
def kernel(mossy_input, connectivity_mask, weights, threshold):
    u2 = mossy_input.reshape(1, N_MOSSY)
    thr = threshold.reshape(1)

    out = pl.pallas_call(
        _body,
        grid=(GRID,),
        in_specs=[
            pl.BlockSpec((1, N_MOSSY), lambda b: (0, 0)),
            pl.BlockSpec((RB, N_MOSSY), lambda b: (b, 0)),
            pl.BlockSpec((RB, N_MOSSY), lambda b: (b, 0)),
            pl.BlockSpec(memory_space=pltpu.SMEM),
        ],
        out_specs=pl.BlockSpec((GRID, RB), lambda b: (0, 0)),
        out_shape=jax.ShapeDtypeStruct((GRID, RB), jnp.float32),
        compiler_params=pltpu.CompilerParams(
            dimension_semantics=("arbitrary",),
        ),
    )(u2, weights, connectivity_mask, thr)
    return out.reshape(N_GRANULE)


# speedup vs baseline: 1.1119x; 1.0112x over previous
import jax
import jax.numpy as jnp
import numpy as np
from jax import lax
from jax.experimental import pallas as pl
from jax.experimental.pallas import tpu as pltpu

N_MOSSY = 1024
N_GRANULE = 65536
K = int(N_GRANULE * 0.02)

RB = 2048
GRID = N_GRANULE // RB
INT_MIN = np.int32(-(2 ** 31))


def _sortable_key(y):
    s = lax.bitcast_convert_type(y, jnp.int32)
    return jnp.where(s < 0, ~(s ^ INT_MIN), s)


def _body(u_ref, w_ref, m_ref, thr_ref, out_ref):
    b = pl.program_id(0)
    masked = w_ref[...] * m_ref[...]
    y_blk = lax.dot_general(u_ref[...], masked, (((1,), (1,)), ((), ())),
                            preferred_element_type=jnp.float32)
    out_ref[pl.ds(b, 1), :] = y_blk

    @pl.when(b == GRID - 1)
    def _select():
        y = out_ref[...]
        key = _sortable_key(y)

        def bit_step(i, t_u):
            hi = jnp.int32(1) << (31 - 2 * i)
            lo = jnp.int32(1) << (30 - 2 * i)
            c1 = t_u | hi
            c2 = t_u | lo
            c3 = c1 | lo
            n1 = jnp.sum((key >= (c1 ^ INT_MIN)).astype(jnp.int32))
            n2 = jnp.sum((key >= (c2 ^ INT_MIN)).astype(jnp.int32))
            n3 = jnp.sum((key >= (c3 ^ INT_MIN)).astype(jnp.int32))
            return jnp.where(n1 >= K,
                             jnp.where(n3 >= K, c3, c1),
                             jnp.where(n2 >= K, c2, t_u))

        t_u = lax.fori_loop(0, 16, bit_step, jnp.int32(0))
        t = t_u ^ INT_MIN

        gt = key > t
        eq = key == t
        c_gt = jnp.sum(gt.astype(jnp.int32))
        c_eq = jnp.sum(eq.astype(jnp.int32))
        m = K - c_gt

        ri = lax.broadcasted_iota(jnp.int32, (GRID, RB), 0)
        ci = lax.broadcasted_iota(jnp.int32, (GRID, RB), 1)
        idx = ri * RB + ci

        def _all_ties(_):
            return jnp.int32(N_GRANULE)

        def _search_ties(_):
            def idx_step(i, x):
                cand = x | (jnp.int32(1) << (16 - i))
                c = jnp.sum((eq & (idx < cand)).astype(jnp.int32))
                return jnp.where(c < m, cand, x)

            return lax.fori_loop(0, 17, idx_step, jnp.int32(0))

        x = lax.cond(c_eq == m, _all_ties, _search_ties, jnp.int32(0))
        sel = gt | (eq & (idx < x + 1))

        thr = thr_ref[0]
        out_ref[...] = jnp.where(sel, jnp.maximum(y - thr, 0.0), 0.0)


@jax.jit
def kernel(mossy_input, connectivity_mask, weights, threshold):
    u2 = mossy_input.reshape(1, N_MOSSY)
    thr = threshold.reshape(1)

    out = pl.pallas_call(
        _body,
        grid=(GRID,),
        in_specs=[
            pl.BlockSpec((1, N_MOSSY), lambda b: (0, 0)),
            pl.BlockSpec((RB, N_MOSSY), lambda b: (b, 0)),
            pl.BlockSpec((RB, N_MOSSY), lambda b: (b, 0)),
            pl.BlockSpec(memory_space=pltpu.SMEM),
        ],
        out_specs=pl.BlockSpec((GRID, RB), lambda b: (0, 0)),
        out_shape=jax.ShapeDtypeStruct((GRID, RB), jnp.float32),
        compiler_params=pltpu.CompilerParams(
            dimension_semantics=("arbitrary",),
        ),
    )(u2, weights, connectivity_mask, thr)
    return out.reshape(N_GRANULE)
